# baseline (speedup 1.0000x reference)
import functools

import jax
import jax.numpy as jnp
from jax.experimental import pallas as pl

_LATENT_DIM = 64
_NUM_CODES = 1024
_BLOCK_ROWS = 512


def _vq_body(x_ref, emb_ref, o_ref):
    xb = x_ref[...]
    emb = emb_ref[...]
    sim = jnp.dot(xb, emb, preferred_element_type=jnp.float32)
    e2 = jnp.sum(emb * emb, axis=0, keepdims=True)
    scores = e2 - 2.0 * sim
    idx = jnp.argmin(scores, axis=1).reshape(-1, 1)
    col = jax.lax.broadcasted_iota(jnp.int32, scores.shape, 1)
    onehot = (col == idx).astype(jnp.float32)
    o_ref[...] = jax.lax.dot_general(
        onehot, emb, (((1,), (1,)), ((), ())),
        preferred_element_type=jnp.float32)


@functools.partial(jax.jit, static_argnames=("interpret",))
def kernel(x, embeddings, interpret=False):
    orig_shape = x.shape
    xf = x.reshape(-1, _LATENT_DIM)
    rows = xf.shape[0]
    grid = (rows // _BLOCK_ROWS,)
    out = pl.pallas_call(
        _vq_body,
        grid=grid,
        in_specs=[
            pl.BlockSpec((_BLOCK_ROWS, _LATENT_DIM), lambda i: (i, 0)),
            pl.BlockSpec((_LATENT_DIM, _NUM_CODES), lambda i: (0, 0)),
        ],
        out_specs=pl.BlockSpec((_BLOCK_ROWS, _LATENT_DIM), lambda i: (i, 0)),
        out_shape=jax.ShapeDtypeStruct((rows, _LATENT_DIM), jnp.float32),
        interpret=interpret,
    )(xf, embeddings)
    return out.reshape(orig_shape)


# speedup vs baseline: 1.1824x; 1.1824x over previous
import functools

import jax
import jax.numpy as jnp
from jax.experimental import pallas as pl

_LATENT_DIM = 64
_NUM_CODES = 1024
_BLOCK_ROWS = 2048


def _vq_body(x_ref, emb_ref, o_ref):
    xb = x_ref[...]
    emb = emb_ref[...]
    sim = jnp.dot(xb, emb, preferred_element_type=jnp.float32)
    e2 = jnp.sum(emb * emb, axis=0, keepdims=True)
    scores = e2 - 2.0 * sim
    idx = jnp.argmin(scores, axis=1).reshape(-1, 1)
    col = jax.lax.broadcasted_iota(jnp.int32, scores.shape, 1)
    onehot = (col == idx).astype(jnp.float32)
    o_ref[...] = jax.lax.dot_general(
        onehot, emb, (((1,), (1,)), ((), ())),
        preferred_element_type=jnp.float32)


@functools.partial(jax.jit, static_argnames=("interpret",))
def kernel(x, embeddings, interpret=False):
    orig_shape = x.shape
    xf = x.reshape(-1, _LATENT_DIM)
    rows = xf.shape[0]
    grid = (rows // _BLOCK_ROWS,)
    out = pl.pallas_call(
        _vq_body,
        grid=grid,
        in_specs=[
            pl.BlockSpec((_BLOCK_ROWS, _LATENT_DIM), lambda i: (i, 0)),
            pl.BlockSpec((_LATENT_DIM, _NUM_CODES), lambda i: (0, 0)),
        ],
        out_specs=pl.BlockSpec((_BLOCK_ROWS, _LATENT_DIM), lambda i: (i, 0)),
        out_shape=jax.ShapeDtypeStruct((rows, _LATENT_DIM), jnp.float32),
        interpret=interpret,
    )(xf, embeddings)
    return out.reshape(orig_shape)


# speedup vs baseline: 1.2125x; 1.0254x over previous
import functools

import jax
import jax.numpy as jnp
from jax.experimental import pallas as pl

_LATENT_DIM = 64
_NUM_CODES = 1024
_BLOCK_ROWS = 4096


def _vq_body(x_ref, emb_ref, o_ref):
    xb = x_ref[...]
    emb = emb_ref[...]
    sim = jnp.dot(xb, emb, preferred_element_type=jnp.float32)
    e2 = jnp.sum(emb * emb, axis=0, keepdims=True)
    scores = e2 - 2.0 * sim
    idx = jnp.argmin(scores, axis=1).reshape(-1, 1)
    col = jax.lax.broadcasted_iota(jnp.int32, scores.shape, 1)
    onehot = (col == idx).astype(jnp.float32)
    o_ref[...] = jax.lax.dot_general(
        onehot, emb, (((1,), (1,)), ((), ())),
        preferred_element_type=jnp.float32)


@functools.partial(jax.jit, static_argnames=("interpret",))
def kernel(x, embeddings, interpret=False):
    orig_shape = x.shape
    xf = x.reshape(-1, _LATENT_DIM)
    rows = xf.shape[0]
    grid = (rows // _BLOCK_ROWS,)
    out = pl.pallas_call(
        _vq_body,
        grid=grid,
        in_specs=[
            pl.BlockSpec((_BLOCK_ROWS, _LATENT_DIM), lambda i: (i, 0)),
            pl.BlockSpec((_LATENT_DIM, _NUM_CODES), lambda i: (0, 0)),
        ],
        out_specs=pl.BlockSpec((_BLOCK_ROWS, _LATENT_DIM), lambda i: (i, 0)),
        out_shape=jax.ShapeDtypeStruct((rows, _LATENT_DIM), jnp.float32),
        interpret=interpret,
    )(xf, embeddings)
    return out.reshape(orig_shape)
